# baseline (speedup 1.0000x reference)
import functools

import jax
import jax.numpy as jnp
from jax import lax
from jax.experimental import pallas as pl
from jax.experimental.pallas import tpu as pltpu
from jax.experimental.pallas import tpu_sc as plsc

_VOCAB = 1000000
_EMBED = 64
_B = 4096 * 200

_NC = 2
_NS = 16
_NW = _NC * _NS

_CHUNK = 128
_B_PER_W = _B // _NW
_CHUNKS_PER_W = _B_PER_W // _CHUNK


def _gather_body(table_hbm, idx_hbm, out_hbm, idx_v, rows_v, sem):
  wid = lax.axis_index("s") * _NC + lax.axis_index("c")
  base_w = wid * _B_PER_W

  def step(c, carry):
    base = base_w + c * _CHUNK
    pltpu.sync_copy(idx_hbm.at[pl.ds(base, _CHUNK)], idx_v)
    pltpu.async_copy(table_hbm.at[idx_v], rows_v, sem).wait()
    pltpu.sync_copy(rows_v, out_hbm.at[pl.ds(base, _CHUNK)])
    return carry

  lax.fori_loop(0, _CHUNKS_PER_W, step, 0)


@jax.jit
def kernel(sentence, W_word):
  idx = sentence.reshape(_B).astype(jnp.int32)
  mesh = plsc.VectorSubcoreMesh(core_axis_name="c", subcore_axis_name="s")
  out = pl.kernel(
      _gather_body,
      out_type=jax.ShapeDtypeStruct((_B, _EMBED), jnp.float32),
      mesh=mesh,
      scratch_types=[
          pltpu.VMEM((_CHUNK,), jnp.int32),
          pltpu.VMEM((_CHUNK, _EMBED), jnp.float32),
          pltpu.SemaphoreType.DMA,
      ],
      compiler_params=pltpu.CompilerParams(use_tc_tiling_on_sc=False),
  )(W_word, idx)
  return out.reshape(sentence.shape[0], sentence.shape[1], _EMBED)


# speedup vs baseline: 1.1950x; 1.1950x over previous
import jax
import jax.numpy as jnp
from jax import lax
from jax.experimental import pallas as pl
from jax.experimental.pallas import tpu as pltpu
from jax.experimental.pallas import tpu_sc as plsc

_VOCAB = 1000000
_EMBED = 64
_B = 4096 * 200

_NC = 2
_NS = 16
_NW = _NC * _NS

_CHUNK = 128
_GPB = 4
_SUPER = _CHUNK * _GPB
_B_PER_W = _B // _NW
_CHUNKS_PER_W = _B_PER_W // _CHUNK
_SUPERS_PER_W = _B_PER_W // _SUPER


def _body(table_hbm, idx_hbm, out_hbm, idx_v, rows0, rows1, sem_g0, sem_g1):
  wid = lax.axis_index("s") * _NC + lax.axis_index("c")
  base_w = wid * _B_PER_W

  pltpu.sync_copy(idx_hbm.at[wid], idx_v)

  rows = (rows0, rows1)
  sems = (sem_g0, sem_g1)

  def fire(b, s):
    for j in range(_GPB):
      pltpu.async_copy(
          table_hbm.at[idx_v.at[s * _GPB + j]],
          rows[b].at[pl.ds(j * _CHUNK, _CHUNK)],
          sems[b],
      )

  def drain(b):
    pltpu.make_async_copy(
        out_hbm.at[pl.ds(0, _SUPER)], rows[b], sems[b]).wait()

  def store(b, s):
    pltpu.sync_copy(rows[b], out_hbm.at[pl.ds(base_w + s * _SUPER, _SUPER)])

  fire(0, 0)
  fire(1, 1)

  def step(s2, carry):
    for b in range(2):
      s = s2 * 2 + b
      drain(b)
      store(b, s)
      fire(b, s + 2)
    return carry

  lax.fori_loop(0, _SUPERS_PER_W // 2 - 1, step, 0)

  for b in range(2):
    s = _SUPERS_PER_W - 2 + b
    drain(b)
    store(b, s)


@jax.jit
def kernel(sentence, W_word):
  idx = sentence.reshape(_NW, _CHUNKS_PER_W, _CHUNK).astype(jnp.int32)
  mesh = plsc.VectorSubcoreMesh(core_axis_name="c", subcore_axis_name="s")
  out = pl.kernel(
      _body,
      out_type=jax.ShapeDtypeStruct((_B, _EMBED), jnp.float32),
      mesh=mesh,
      scratch_types=[
          pltpu.VMEM((_CHUNKS_PER_W, _CHUNK), jnp.int32),
          pltpu.VMEM((_SUPER, _EMBED), jnp.float32),
          pltpu.VMEM((_SUPER, _EMBED), jnp.float32),
          pltpu.SemaphoreType.DMA,
          pltpu.SemaphoreType.DMA,
      ],
      compiler_params=pltpu.CompilerParams(use_tc_tiling_on_sc=False),
  )(W_word, idx)
  return out.reshape(sentence.shape[0], sentence.shape[1], _EMBED)


# speedup vs baseline: 1.2289x; 1.0284x over previous
import jax
import jax.numpy as jnp
from jax import lax
from jax.experimental import pallas as pl
from jax.experimental.pallas import tpu as pltpu
from jax.experimental.pallas import tpu_sc as plsc

_VOCAB = 1000000
_EMBED = 64
_S = 4096
_T = 200
_B = _S * _T

_NC = 2
_NS = 16
_NW = _NC * _NS

_CHUNK = 128
_GPB = 4
_SUPER = _CHUNK * _GPB
_B_PER_W = _B // _NW
_CHUNKS_PER_W = _B_PER_W // _CHUNK
_SUPERS_PER_W = _B_PER_W // _SUPER
_NCHUNK = _B // _CHUNK


def _body(table_hbm, idx_hbm, out_hbm, idx_v, rows0, rows1, sem_g0, sem_g1):
  wid = lax.axis_index("s") * _NC + lax.axis_index("c")
  base_c = wid * _CHUNKS_PER_W

  pltpu.sync_copy(idx_hbm.at[wid], idx_v)

  rows = (rows0, rows1)
  sems = (sem_g0, sem_g1)

  def fire(b, s):
    for j in range(_GPB):
      pltpu.async_copy(
          table_hbm.at[idx_v.at[s * _GPB + j]],
          rows[b].at[j],
          sems[b],
      )

  def drain(b):
    for j in range(_GPB):
      pltpu.make_async_copy(
          out_hbm.at[0, pl.ds(0, _CHUNK)], rows[b].at[j], sems[b]).wait()

  def store(b, s):
    for j in range(_GPB):
      c = base_c + s * _GPB + j
      t = c // (_S // _CHUNK)
      s0 = (c % (_S // _CHUNK)) * _CHUNK
      pltpu.sync_copy(rows[b].at[j], out_hbm.at[t, pl.ds(s0, _CHUNK)])

  fire(0, 0)
  fire(1, 1)

  def step(s2, carry):
    for b in range(2):
      s = s2 * 2 + b
      drain(b)
      store(b, s)
      fire(b, s + 2)
    return carry

  lax.fori_loop(0, _SUPERS_PER_W // 2 - 1, step, 0)

  for b in range(2):
    s = _SUPERS_PER_W - 2 + b
    drain(b)
    store(b, s)


@jax.jit
def kernel(sentence, W_word):
  idx = sentence.T.astype(jnp.int32).reshape(_NW, _CHUNKS_PER_W, _CHUNK)
  mesh = plsc.VectorSubcoreMesh(core_axis_name="c", subcore_axis_name="s")
  out = pl.kernel(
      _body,
      out_type=jax.ShapeDtypeStruct((_T, _S, _EMBED), jnp.float32),
      mesh=mesh,
      scratch_types=[
          pltpu.VMEM((_B_PER_W // _CHUNK, _CHUNK), jnp.int32),
          pltpu.VMEM((_GPB, _CHUNK, _EMBED), jnp.float32),
          pltpu.VMEM((_GPB, _CHUNK, _EMBED), jnp.float32),
          pltpu.SemaphoreType.DMA,
          pltpu.SemaphoreType.DMA,
      ],
      compiler_params=pltpu.CompilerParams(use_tc_tiling_on_sc=False),
  )(W_word, idx)
  return out.swapaxes(0, 1)
